# baseline (speedup 1.0000x reference)
import functools

import jax
import jax.numpy as jnp
from jax import lax
from jax.experimental import pallas as pl
from jax.experimental.pallas import tpu as pltpu
from jax.experimental.pallas import tpu_sc as plsc

CH = 2048
NB = 4
GRP = 16


def kernel(segment_ids, table):
    batch, num_fields = segment_ids.shape
    num_rows, d_model = table.shape
    ids_t = segment_ids.astype(jnp.int32).T.reshape(-1)
    table_t = table.T

    info = plsc.get_sparse_core_info()
    num_workers = info.num_cores * info.num_subcores

    chunks_per_f = batch // CH
    total = num_fields * chunks_per_f
    num_groups = total // GRP
    tile_rows = num_fields * (d_model // 8) * (batch // 128)

    mesh = plsc.VectorSubcoreMesh(core_axis_name="c", subcore_axis_name="s")

    @functools.partial(
        pl.kernel,
        out_type=jax.ShapeDtypeStruct((tile_rows, 8, 128), jnp.float32),
        mesh=mesh,
        scratch_types=(
            [pltpu.VMEM_SHARED((2, GRP, CH), jnp.int32),
             pltpu.VMEM((num_rows,), jnp.float32),
             pltpu.VMEM((NB, CH), jnp.int32),
             pltpu.VMEM((NB, CH // 128, 1, 128), jnp.float32),
             pltpu.SemaphoreType.DMA,
             pltpu.SemaphoreType.DMA]
            + [pltpu.SemaphoreType.DMA] * (2 * NB)
        ),
        compiler_params=pltpu.CompilerParams(
            use_tc_tiling_on_sc=False, needs_layout_passes=False),
    )
    def gather_kernel(ids_hbm, tab_hbm, out_hbm, ids_spm, trow, ids_v,
                      vals_v, sem_t, sem_p, *sems):
        sem_i = sems[:NB]
        sem_o = sems[NB:]
        sid = lax.axis_index("s")
        wid = sid * info.num_cores + lax.axis_index("c")
        t_d = wid // 8
        s_sub = wid % 8

        def p_desc(g1):
            return pltpu.make_async_copy(
                ids_hbm.at[pl.ds((g1 * GRP + sid) * CH, CH)],
                ids_spm.at[lax.rem(g1, 2), sid],
                sem_p)

        def i_desc(g, k, sl):
            return pltpu.make_async_copy(
                ids_spm.at[lax.rem(g, 2), lax.rem(k + sid, GRP)],
                ids_v.at[sl],
                sem_i[sl])

        def o_desc(g, k, sl):
            c = g * GRP + lax.rem(k + sid, GRP)
            f = c // chunks_per_f
            cc = c % chunks_per_f
            r0 = f * (d_model // 8) * (batch // 128) + t_d * (batch // 128) \
                + cc * (CH // 128)
            return pltpu.make_async_copy(
                vals_v.at[sl],
                out_hbm.at[pl.ds(r0, CH // 128), pl.ds(s_sub, 1)],
                sem_o[sl])

        def compute(sl):
            iv = ids_v.at[sl]
            vv = vals_v.at[sl]

            @plsc.parallel_loop(0, CH // 16, step=1, unroll=8)
            def cbody(j):
                idx = iv[pl.ds(j * 16, 16)]
                vals = plsc.load_gather(trow, [idx])
                vv[j // 8, 0, pl.ds((j % 8) * 16, 16)] = vals

        def group_body(g, first, last):
            if not last:
                p_desc(g + 1).start()
            for k in range(NB):
                i_desc(g, k, k).start()
            for k in range(GRP):
                sl = k % NB
                i_desc(g, k, sl).wait()
                if not (first and k < NB):
                    o_desc(g, k, sl).wait()
                compute(sl)
                o_desc(g, k, sl).start()
                if k < GRP - NB:
                    i_desc(g, k + NB, sl).start()
            if not last:
                p_desc(g + 1).wait()
            plsc.subcore_barrier()

        pltpu.make_async_copy(tab_hbm.at[wid], trow, sem_t).start()
        p_desc(0).start()
        p_desc(0).wait()
        plsc.subcore_barrier()
        pltpu.make_async_copy(tab_hbm.at[wid], trow, sem_t).wait()

        group_body(0, True, False)

        def body(g, carry):
            group_body(g, False, False)
            return carry

        lax.fori_loop(1, num_groups - 1, body, 0)
        group_body(num_groups - 1, False, True)
        for sl in range(NB):
            o_desc(num_groups - 1, GRP - NB + sl, sl).wait()

    out_lin = gather_kernel(ids_t, table_t)
    x = out_lin.reshape(num_fields, d_model // 8, batch // 128, 8, 128)
    y = x.transpose(2, 4, 0, 1, 3)
    return y.reshape(batch, num_fields, d_model)


# speedup vs baseline: 1.0011x; 1.0011x over previous
import functools

import jax
import jax.numpy as jnp
from jax import lax
from jax.experimental import pallas as pl
from jax.experimental.pallas import tpu as pltpu
from jax.experimental.pallas import tpu_sc as plsc

CH = 2048
NB = 4
GRP = 16


def kernel(segment_ids, table):
    batch, num_fields = segment_ids.shape
    num_rows, d_model = table.shape
    ids_t = segment_ids.astype(jnp.int32).T.reshape(-1)
    table_t = table.T

    info = plsc.get_sparse_core_info()
    num_workers = info.num_cores * info.num_subcores

    chunks_per_f = batch // CH
    total = num_fields * chunks_per_f
    num_groups = total // GRP
    tile_rows = num_fields * (d_model // 8) * (batch // 128)

    mesh = plsc.VectorSubcoreMesh(core_axis_name="c", subcore_axis_name="s")

    @functools.partial(
        pl.kernel,
        out_type=jax.ShapeDtypeStruct((tile_rows, 8, 128), jnp.float32),
        mesh=mesh,
        scratch_types=(
            [pltpu.VMEM_SHARED((2, GRP, CH), jnp.int32),
             pltpu.VMEM((num_rows,), jnp.float32),
             pltpu.VMEM((NB, CH), jnp.int32),
             pltpu.VMEM((NB, CH // 128, 1, 128), jnp.float32),
             pltpu.SemaphoreType.DMA,
             pltpu.SemaphoreType.DMA]
            + [pltpu.SemaphoreType.DMA] * (2 * NB)
        ),
        compiler_params=pltpu.CompilerParams(
            use_tc_tiling_on_sc=False, needs_layout_passes=False),
    )
    def gather_kernel(ids_hbm, tab_hbm, out_hbm, ids_spm, trow, ids_v,
                      vals_v, sem_t, sem_p, *sems):
        sem_i = sems[:NB]
        sem_o = sems[NB:]
        sid = lax.axis_index("s")
        wid = sid * info.num_cores + lax.axis_index("c")
        t_d = wid // 8
        s_sub = wid % 8

        def p_desc(g1):
            return pltpu.make_async_copy(
                ids_hbm.at[pl.ds((g1 * GRP + sid) * CH, CH)],
                ids_spm.at[lax.rem(g1, 2), sid],
                sem_p)

        def i_desc(g, k, sl):
            return pltpu.make_async_copy(
                ids_spm.at[lax.rem(g, 2), lax.rem(k + sid, GRP)],
                ids_v.at[sl],
                sem_i[sl])

        def o_desc(g, k, sl):
            c = g * GRP + lax.rem(k + sid, GRP)
            f = c // chunks_per_f
            cc = c % chunks_per_f
            r0 = f * (d_model // 8) * (batch // 128) + t_d * (batch // 128) \
                + cc * (CH // 128)
            return pltpu.make_async_copy(
                vals_v.at[sl],
                out_hbm.at[pl.ds(r0, CH // 128), pl.ds(s_sub, 1)],
                sem_o[sl])

        def compute(sl):
            iv = ids_v.at[sl]
            vv = vals_v.at[sl]

            @plsc.parallel_loop(0, CH // 16, step=1, unroll=16)
            def cbody(j):
                idx = iv[pl.ds(j * 16, 16)]
                vals = plsc.load_gather(trow, [idx])
                vv[j // 8, 0, pl.ds((j % 8) * 16, 16)] = vals

        def group_body(g, first, last):
            if not last:
                p_desc(g + 1).start()
            for k in range(NB):
                i_desc(g, k, k).start()
            for k in range(GRP):
                sl = k % NB
                i_desc(g, k, sl).wait()
                if not (first and k < NB):
                    o_desc(g, k, sl).wait()
                compute(sl)
                o_desc(g, k, sl).start()
                if k < GRP - NB:
                    i_desc(g, k + NB, sl).start()
            if not last:
                p_desc(g + 1).wait()
            plsc.subcore_barrier()

        pltpu.make_async_copy(tab_hbm.at[wid], trow, sem_t).start()
        p_desc(0).start()
        p_desc(0).wait()
        plsc.subcore_barrier()
        pltpu.make_async_copy(tab_hbm.at[wid], trow, sem_t).wait()

        group_body(0, True, False)

        def body(g, carry):
            group_body(g, False, False)
            return carry

        lax.fori_loop(1, num_groups - 1, body, 0)
        group_body(num_groups - 1, False, True)
        for sl in range(NB):
            o_desc(num_groups - 1, GRP - NB + sl, sl).wait()

    out_lin = gather_kernel(ids_t, table_t)
    x = out_lin.reshape(num_fields, d_model // 8, batch // 128, 8, 128)
    y = x.transpose(2, 4, 0, 1, 3)
    return y.reshape(batch, num_fields, d_model)


# speedup vs baseline: 1.0359x; 1.0348x over previous
import functools

import jax
import jax.numpy as jnp
from jax import lax
from jax.experimental import pallas as pl
from jax.experimental.pallas import tpu as pltpu
from jax.experimental.pallas import tpu_sc as plsc

CH = 4096
NB = 2
GRP = 16


def kernel(segment_ids, table):
    batch, num_fields = segment_ids.shape
    num_rows, d_model = table.shape
    ids_t = segment_ids.astype(jnp.int32).T.reshape(-1)
    table_t = table.T

    info = plsc.get_sparse_core_info()
    num_workers = info.num_cores * info.num_subcores

    chunks_per_f = batch // CH
    total = num_fields * chunks_per_f
    num_groups = total // GRP
    tile_rows = num_fields * (d_model // 8) * (batch // 128)

    mesh = plsc.VectorSubcoreMesh(core_axis_name="c", subcore_axis_name="s")

    @functools.partial(
        pl.kernel,
        out_type=jax.ShapeDtypeStruct((tile_rows, 8, 128), jnp.float32),
        mesh=mesh,
        scratch_types=(
            [pltpu.VMEM_SHARED((2, GRP, CH), jnp.int32),
             pltpu.VMEM((num_rows,), jnp.float32),
             pltpu.VMEM((NB, CH), jnp.int32),
             pltpu.VMEM((NB, CH // 128, 1, 128), jnp.float32),
             pltpu.SemaphoreType.DMA,
             pltpu.SemaphoreType.DMA]
            + [pltpu.SemaphoreType.DMA] * (2 * NB)
        ),
        compiler_params=pltpu.CompilerParams(
            use_tc_tiling_on_sc=False, needs_layout_passes=False),
    )
    def gather_kernel(ids_hbm, tab_hbm, out_hbm, ids_spm, trow, ids_v,
                      vals_v, sem_t, sem_p, *sems):
        sem_i = sems[:NB]
        sem_o = sems[NB:]
        sid = lax.axis_index("s")
        wid = sid * info.num_cores + lax.axis_index("c")
        t_d = wid // 8
        s_sub = wid % 8

        def p_desc(g1):
            return pltpu.make_async_copy(
                ids_hbm.at[pl.ds((g1 * GRP + sid) * CH, CH)],
                ids_spm.at[lax.rem(g1, 2), sid],
                sem_p)

        def i_desc(g, k, sl):
            return pltpu.make_async_copy(
                ids_spm.at[lax.rem(g, 2), lax.rem(k + sid, GRP)],
                ids_v.at[sl],
                sem_i[sl])

        def o_desc(g, k, sl):
            c = g * GRP + lax.rem(k + sid, GRP)
            f = c // chunks_per_f
            cc = c % chunks_per_f
            r0 = f * (d_model // 8) * (batch // 128) + t_d * (batch // 128) \
                + cc * (CH // 128)
            return pltpu.make_async_copy(
                vals_v.at[sl],
                out_hbm.at[pl.ds(r0, CH // 128), pl.ds(s_sub, 1)],
                sem_o[sl])

        def compute(sl):
            iv = ids_v.at[sl]
            vv = vals_v.at[sl]

            @plsc.parallel_loop(0, CH // 16, step=1, unroll=16)
            def cbody(j):
                idx = iv[pl.ds(j * 16, 16)]
                vals = plsc.load_gather(trow, [idx])
                vv[j // 8, 0, pl.ds((j % 8) * 16, 16)] = vals

        def group_body(g, first, last):
            if not last:
                p_desc(g + 1).start()
            for k in range(NB):
                i_desc(g, k, k).start()
            for k in range(GRP):
                sl = k % NB
                i_desc(g, k, sl).wait()
                if not (first and k < NB):
                    o_desc(g, k, sl).wait()
                compute(sl)
                o_desc(g, k, sl).start()
                if k < GRP - NB:
                    i_desc(g, k + NB, sl).start()
            if not last:
                p_desc(g + 1).wait()
            plsc.subcore_barrier()

        pltpu.make_async_copy(tab_hbm.at[wid], trow, sem_t).start()
        p_desc(0).start()
        p_desc(0).wait()
        plsc.subcore_barrier()
        pltpu.make_async_copy(tab_hbm.at[wid], trow, sem_t).wait()

        group_body(0, True, False)

        def body(g, carry):
            group_body(g, False, False)
            return carry

        lax.fori_loop(1, num_groups - 1, body, 0)
        group_body(num_groups - 1, False, True)
        for sl in range(NB):
            o_desc(num_groups - 1, GRP - NB + sl, sl).wait()

    out_lin = gather_kernel(ids_t, table_t)
    x = out_lin.reshape(num_fields, d_model // 8, batch // 128, 8, 128)
    y = x.transpose(2, 4, 0, 1, 3)
    return y.reshape(batch, num_fields, d_model)
